# baseline (speedup 1.0000x reference)
import functools

import jax
import jax.numpy as jnp
from jax import lax
from jax.experimental import pallas as pl
from jax.experimental.pallas import tpu as pltpu
from jax.experimental.pallas import tpu_sc as plsc

L = 16
NC = 2
NS = 16
NW = NC * NS
CH = 128

def _mesh():
  return plsc.VectorSubcoreMesh(core_axis_name="c", subcore_axis_name="s")


def _zero_vmem_2d(ref, rows, cols):
  def row_body(r, _):
    for k in range(cols // L):
      ref[r, pl.ds(k * L, L)] = jnp.zeros((L,), jnp.float32)
    return ()
  lax.fori_loop(0, rows, row_body, ())


def _make_deg_kernel(n_pad, nch, nproc):
  rpt = n_pad // NS

  @functools.partial(
      pl.kernel,
      mesh=_mesh(),
      out_type=jax.ShapeDtypeStruct((NC * n_pad,), jnp.float32),
      scratch_types=[
          pltpu.VMEM((nch, CH), jnp.int32),
          pltpu.VMEM((CH,), jnp.float32),
          pltpu.VMEM((rpt,), jnp.float32),
          pltpu.VMEM_SHARED((n_pad,), jnp.float32),
          pltpu.SemaphoreType.DMA,
      ],
  )
  def deg_kernel(dst_hbm, out_hbm, dst_v, ones_v, zero_v, acc, sem):
    c = lax.axis_index("c")
    s = lax.axis_index("s")
    wid = s * NC + c
    pltpu.sync_copy(dst_hbm.at[pl.ds(wid * nch, nch)], dst_v)
    for k in range(CH // L):
      ones_v[pl.ds(k * L, L)] = jnp.ones((L,), jnp.float32)
    def zbody(i, _):
      zero_v[pl.ds(i * L, L)] = jnp.zeros((L,), jnp.float32)
      return ()
    lax.fori_loop(0, rpt // L, zbody, ())
    pltpu.sync_copy(zero_v, acc.at[pl.ds(s * rpt, rpt)])
    plsc.subcore_barrier()

    def body(j, _):
      pltpu.async_copy(ones_v, acc.at[dst_v.at[j]], sem, add=True)
      return ()
    lax.fori_loop(0, nproc, body, ())
    def drain(j, _):
      pltpu.make_async_copy(ones_v, acc.at[dst_v.at[j]], sem).wait()
      return ()
    lax.fori_loop(0, nproc, drain, ())

    plsc.subcore_barrier()
    pltpu.sync_copy(acc.at[pl.ds(s * rpt, rpt)],
                    out_hbm.at[pl.ds(c * n_pad + s * rpt, rpt)])

  return deg_kernel


def _make_agg_kernel(n_pad, d, nch, nproc):
  rpt = n_pad // NS

  @functools.partial(
      pl.kernel,
      mesh=_mesh(),
      out_type=jax.ShapeDtypeStruct((NC, n_pad, d), jnp.float32),
      scratch_types=[
          pltpu.VMEM((CH, d), jnp.float32),
          pltpu.VMEM((CH, d), jnp.float32),
          pltpu.VMEM((((nproc // 2 + 7) // 8) * 8, CH), jnp.int32),
          pltpu.VMEM((((nproc // 2 + 7) // 8) * 8, CH), jnp.int32),
          pltpu.VMEM_SHARED((n_pad, d), jnp.float32),
          pltpu.SemaphoreType.DMA,
          pltpu.SemaphoreType.DMA,
      ],
  )
  def agg_kernel(g_hbm, src_hbm, dst_hbm, out_hbm,
                 buf0, buf1, sidx_v, didx_v, acc, gs0, gs1):
    HH = CH // 2
    c = lax.axis_index("c")
    s = lax.axis_index("s")
    wid = s * NC + c
    base = wid * nch
    nph0 = ((nproc // 2 + 7) // 8) * 8
    procs = (nph0, nproc - nph0)
    stages = (nph0, ((nproc - nph0 + 7) // 8) * 8)
    bufs = (buf0, buf1)
    gsems = (gs0, gs1)

    _zero_vmem_2d(buf0, CH, d)
    for k in range(rpt // CH):
      pltpu.sync_copy(buf0, acc.at[pl.ds(s * rpt + k * CH, CH)])

    for p in range(2):
      nph = procs[p]
      nst = stages[p]
      poff = base + p * procs[0]
      pltpu.sync_copy(src_hbm.at[pl.ds(poff, nst)],
                      sidx_v.at[pl.ds(0, nst)])
      pltpu.sync_copy(dst_hbm.at[pl.ds(poff, nst)],
                      didx_v.at[pl.ds(0, nst)])
      def start_gather(j, t):
        pltpu.async_copy(g_hbm.at[sidx_v.at[j, pl.ds(0, HH)]],
                         bufs[t].at[pl.ds(0, HH)], gsems[t])
        pltpu.async_copy(g_hbm.at[sidx_v.at[j, pl.ds(HH, HH)]],
                         bufs[t].at[pl.ds(HH, HH)], gsems[t])

      def wait_gather(j, t):
        pltpu.make_async_copy(g_hbm.at[sidx_v.at[j]], bufs[t],
                              gsems[t]).wait()

      start_gather(0, 0)
      start_gather(1, 1)
      if p == 0:
        plsc.subcore_barrier()

      def body(i, _):
        for t in range(2):
          j = 2 * i + t
          wait_gather(j, t)
          pltpu.sync_copy(bufs[t], acc.at[didx_v.at[j]], add=True)
          start_gather(j + 2, t)
        return ()
      lax.fori_loop(0, nph // 2 - 1, body, ())

      for t in range(2):
        j = nph - 2 + t
        wait_gather(j, t)
        pltpu.sync_copy(bufs[t], acc.at[didx_v.at[j]], add=True)

    plsc.subcore_barrier()
    pltpu.sync_copy(acc.at[pl.ds(s * rpt, rpt)],
                    out_hbm.at[c].at[pl.ds(s * rpt, rpt)])

  return agg_kernel


_BM = 1024


def _dinv_scale_body(degp_ref, x_ref, w_ref, db_ref, g_ref):
  nb = _BM // 128
  deg = degp_ref[0] + degp_ref[1]
  dv = lax.rsqrt(jnp.maximum(deg, 1e-12))
  sel = (lax.broadcasted_iota(jnp.int32, (_BM, nb), 0) // 128
         == lax.broadcasted_iota(jnp.int32, (_BM, nb), 1))
  rep = jnp.dot(sel.astype(jnp.float32), dv,
                preferred_element_type=jnp.float32)
  rows = lax.broadcasted_iota(jnp.int32, (_BM, 128), 0)
  cols = lax.broadcasted_iota(jnp.int32, (_BM, 128), 1)
  picked = jnp.where(cols == rows % 128, rep, 0.0)
  dvb = jnp.broadcast_to(jnp.sum(picked, axis=1, keepdims=True),
                         (_BM, 128))
  db_ref[...] = dvb
  h = jnp.dot(x_ref[...], w_ref[...], preferred_element_type=jnp.float32)
  g_ref[...] = h * dvb


def _mid_body(p_ref, db_ref, w_ref, b_ref, o_ref):
  dvb = db_ref[...]
  z = jnp.maximum((p_ref[0] + p_ref[1]) * dvb + b_ref[...], 0.0)
  o_ref[...] = jnp.dot(z, w_ref[...],
                       preferred_element_type=jnp.float32) * dvb


def _final_body(p_ref, db_ref, b_ref, o_ref):
  o_ref[...] = (p_ref[0] + p_ref[1]) * db_ref[...] + b_ref[...]


def _row_spec(d):
  return pl.BlockSpec((_BM, d), lambda i: (i, 0))


def _pair_spec(d):
  return pl.BlockSpec((NC, _BM, d), lambda i: (0, i, 0))


def _full_spec(shape):
  return pl.BlockSpec(shape, lambda i: tuple(0 for _ in shape))


def kernel(x, edge_index, W1, b1, W2, b2):
  n, d = x.shape
  e = edge_index.shape[1]
  assert d == 128

  n_pad = ((n + 2047) // 2048) * 2048
  e2 = e + n
  nproc = (((e2 + NW * CH - 1) // (NW * CH) + 1) // 2) * 2
  nch = ((nproc + 7) // 8) * 8
  e_proc = nproc * NW * CH
  grid = n_pad // _BM

  x_pad = jnp.concatenate([x, jnp.zeros((n_pad - n, d), jnp.float32)], axis=0)
  loop = jnp.arange(n, dtype=jnp.int32)
  n_extra = n_pad - n
  pad_src = jnp.arange(e_proc - e2, dtype=jnp.int32) % n
  pad_dst = n + (jnp.arange(e_proc - e2, dtype=jnp.int32) % n_extra)

  def tile_major(arr3):
    fill = jnp.zeros((NW, nch - nproc, CH), jnp.int32)
    return jnp.concatenate([arr3, fill], axis=1).reshape(NW * nch, CH)

  src2 = tile_major(
      jnp.concatenate([edge_index[0], loop, pad_src]).reshape(NW, nproc, CH))
  dst2 = tile_major(
      jnp.concatenate([edge_index[1], loop, pad_dst]).reshape(NW, nproc, CH))
  b1r = b1.reshape(1, d)
  b2r = b2.reshape(1, d)

  deg_kernel = _make_deg_kernel(n_pad, nch, nproc)
  agg_kernel = _make_agg_kernel(n_pad, d, nch, nproc)

  degp = deg_kernel(dst2)

  dinvb, g1 = pl.pallas_call(
      _dinv_scale_body,
      grid=(grid,),
      in_specs=[pl.BlockSpec((NC, _BM // 128, 128), lambda i: (0, i, 0)),
                _row_spec(d), _full_spec((d, d))],
      out_specs=[_row_spec(d), _row_spec(d)],
      out_shape=[jax.ShapeDtypeStruct((n_pad, d), jnp.float32),
                 jax.ShapeDtypeStruct((n_pad, d), jnp.float32)],
  )(degp.reshape(NC, n_pad // 128, 128), x_pad, W1)

  p1 = agg_kernel(g1, src2, dst2)

  g2 = pl.pallas_call(
      _mid_body,
      grid=(grid,),
      in_specs=[_pair_spec(d), _row_spec(d), _full_spec((d, d)),
                _full_spec((1, d))],
      out_specs=_row_spec(d),
      out_shape=jax.ShapeDtypeStruct((n_pad, d), jnp.float32),
  )(p1, dinvb, W2, b1r)

  p2 = agg_kernel(g2, src2, dst2)

  bmf = 1000
  assert n % bmf == 0 and bmf % 8 == 0
  out = pl.pallas_call(
      _final_body,
      grid=(n // bmf,),
      in_specs=[pl.BlockSpec((NC, bmf, d), lambda i: (0, i, 0)),
                pl.BlockSpec((bmf, d), lambda i: (i, 0)),
                _full_spec((1, d))],
      out_specs=pl.BlockSpec((bmf, d), lambda i: (i, 0)),
      out_shape=jax.ShapeDtypeStruct((n, d), jnp.float32),
  )(p2, dinvb, b2r)

  return out
